# baseline (speedup 1.0000x reference)
import functools

import jax
import jax.numpy as jnp
from jax import lax
from jax.experimental import pallas as pl
from jax.experimental.pallas import tpu as pltpu
from jax.experimental.pallas import tpu_sc as plsc

_NUM_WORDS = 1000000
_D = 64
_DP = 128
_NPAIR = _NUM_WORDS // 2
_B = 4096
_L = 50
_TOTAL = _B * _L
_NC = 2
_NS = 16
_NW = _NC * _NS
_PER_W = _TOTAL // _NW
_STEP = 128
_NSTREAM = _PER_W // _STEP
_RING = 5
_NOUTER = _NSTREAM // _RING

_NTILE = _NUM_WORDS // 8
_PACK_TILES = 2500
_PACK_GRID = _NTILE // _PACK_TILES


def _repack_body(tile_ref, out_ref):
    t3 = tile_ref[...]
    top = t3[:, 0:4, :].reshape(_PACK_TILES * 4, _D)
    bot = t3[:, 4:8, :].reshape(_PACK_TILES * 4, _D)
    out_ref[...] = jnp.concatenate([top, bot], axis=1)


_repack = pl.pallas_call(
    _repack_body,
    grid=(_PACK_GRID,),
    in_specs=[pl.BlockSpec((_PACK_TILES, 8, _D), lambda i: (i, 0, 0))],
    out_specs=pl.BlockSpec((_PACK_TILES * 4, _DP), lambda i: (i, 0)),
    out_shape=jax.ShapeDtypeStruct((_NPAIR, _DP), jnp.float32),
)


@functools.partial(
    pl.kernel,
    mesh=plsc.VectorSubcoreMesh(core_axis_name="c", subcore_axis_name="s"),
    out_type=jax.ShapeDtypeStruct((_TOTAL, _DP), jnp.float32),
    scratch_types=[
        pltpu.VMEM((_PER_W,), jnp.int32),
        pltpu.VMEM((_PER_W,), jnp.int32),
        pltpu.VMEM((_RING, _STEP, _DP), jnp.float32),
        [pltpu.SemaphoreType.DMA] * _RING,
        [pltpu.SemaphoreType.DMA] * _RING,
    ],
)
def _emb_gather(idx_hbm, table_hbm, out_hbm, idx_v, pair_v, bufs, gsems, wsems):
    wid = lax.axis_index("s") * _NC + lax.axis_index("c")
    base = wid * _PER_W
    pltpu.sync_copy(idx_hbm.at[pl.ds(base, _PER_W)], idx_v)

    def pairidx_body(v, carry):
        sl = pl.ds(v * 16, 16)
        pair_v[sl] = lax.shift_right_logical(idx_v[sl], 1)
        return carry

    lax.fori_loop(0, _PER_W // 16, pairidx_body, 0)

    def fire_gather(s, i):
        pltpu.async_copy(
            table_hbm.at[pair_v.at[pl.ds(s * _STEP, _STEP)]],
            bufs.at[i],
            gsems[i],
        )

    for i in range(_RING):
        fire_gather(i, i)

    def outer_body(k, carry):
        for i in range(_RING):
            s = k * _RING + i
            pltpu.make_async_copy(
                table_hbm.at[pair_v.at[pl.ds(s * _STEP, _STEP)]],
                bufs.at[i],
                gsems[i],
            ).wait()
            wcp = pltpu.async_copy(
                bufs.at[i],
                out_hbm.at[pl.ds(base + s * _STEP, _STEP)],
                wsems[i],
            )

            @pl.when(k < _NOUTER - 1)
            def _():
                wcp.wait()
                fire_gather(s + _RING, i)

        return carry

    lax.fori_loop(0, _NOUTER, outer_body, 0)

    for i in range(_RING):
        s = (_NOUTER - 1) * _RING + i
        pltpu.make_async_copy(
            bufs.at[i],
            out_hbm.at[pl.ds(base + s * _STEP, _STEP)],
            wsems[i],
        ).wait()


def kernel(word_ids, word_emb):
    idx = word_ids.reshape(_TOTAL)
    packed = word_emb.reshape(_NPAIR, _DP)
    pairs = _emb_gather(idx, packed)
    hi = (idx & 1)[:, None].astype(bool)
    out = jnp.where(hi, pairs[:, _D:], pairs[:, :_D])
    return out.reshape(_B, _L, _D)


# speedup vs baseline: 1.2489x; 1.2489x over previous
import functools

import jax
import jax.numpy as jnp
from jax import lax
from jax.experimental import pallas as pl
from jax.experimental.pallas import tpu as pltpu
from jax.experimental.pallas import tpu_sc as plsc

_NUM_WORDS = 1000000
_D = 64
_B = 4096
_L = 50
_TOTAL = _B * _L
_NC = 2
_NS = 16
_NW = _NC * _NS
_PER_W = _TOTAL // _NW
_STEP = 128
_NSTREAM = _PER_W // _STEP
_RING = 10
_NOUTER = _NSTREAM // _RING


@functools.partial(
    pl.kernel,
    mesh=plsc.VectorSubcoreMesh(core_axis_name="c", subcore_axis_name="s"),
    out_type=jax.ShapeDtypeStruct((_TOTAL, _D), jnp.float32),
    scratch_types=[
        pltpu.VMEM((_PER_W,), jnp.int32),
        pltpu.VMEM((_PER_W,), jnp.int32),
        pltpu.VMEM((_RING, _STEP, _D), jnp.float32),
        [pltpu.SemaphoreType.DMA] * _RING,
        [pltpu.SemaphoreType.DMA] * _RING,
    ],
    compiler_params=pltpu.CompilerParams(
        use_tc_tiling_on_sc=False, needs_layout_passes=False
    ),
)
def _emb_gather(idx_hbm, table_hbm, out_hbm, idx_v, pair_v, bufs, gsems, wsems):
    wid = lax.axis_index("s") * _NC + lax.axis_index("c")
    base = wid * _PER_W
    out_flat = out_hbm
    pltpu.sync_copy(idx_hbm.at[pl.ds(base, _PER_W)], idx_v)

    def dbl_body(v, carry):
        sl = pl.ds(v * 16, 16)
        pair_v[sl] = idx_v[sl]
        return carry

    lax.fori_loop(0, _PER_W // 16, dbl_body, 0)

    def fire_gather(s, i):
        pltpu.async_copy(
            table_hbm.at[pair_v.at[pl.ds(s * _STEP, _STEP)]],
            bufs.at[i],
            gsems[i],
        )

    for i in range(_RING):
        fire_gather(i, i)

    def outer_body(k, carry):
        for i in range(_RING):
            s = k * _RING + i
            pltpu.make_async_copy(
                table_hbm.at[pair_v.at[pl.ds(s * _STEP, _STEP)]],
                bufs.at[i],
                gsems[i],
            ).wait()
            wcp = pltpu.async_copy(
                bufs.at[i],
                out_flat.at[pl.ds(base + s * _STEP, _STEP)],
                wsems[i],
            )

            @pl.when(k < _NOUTER - 1)
            def _():
                wcp.wait()
                fire_gather(s + _RING, i)

        return carry

    lax.fori_loop(0, _NOUTER, outer_body, 0)

    for i in range(_RING):
        s = (_NOUTER - 1) * _RING + i
        pltpu.make_async_copy(
            bufs.at[i],
            out_flat.at[pl.ds(base + s * _STEP, _STEP)],
            wsems[i],
        ).wait()


def kernel(word_ids, word_emb):
    idx = word_ids.reshape(_TOTAL)
    return _emb_gather(idx, word_emb).reshape(_B, _L, _D)


# speedup vs baseline: 1.2519x; 1.0024x over previous
import functools

import jax
import jax.numpy as jnp
from jax import lax
from jax.experimental import pallas as pl
from jax.experimental.pallas import tpu as pltpu
from jax.experimental.pallas import tpu_sc as plsc

_NUM_WORDS = 1000000
_D = 64
_B = 4096
_L = 50
_TOTAL = _B * _L
_NC = 2
_NS = 16
_NW = _NC * _NS
_PER_W = _TOTAL // _NW
_STEP = 128
_NSTREAM = _PER_W // _STEP
_RING = 10
_NOUTER = _NSTREAM // _RING


@functools.partial(
    pl.kernel,
    mesh=plsc.VectorSubcoreMesh(core_axis_name="c", subcore_axis_name="s"),
    out_type=jax.ShapeDtypeStruct((_TOTAL, _D), jnp.float32),
    scratch_types=[
        pltpu.VMEM((_PER_W,), jnp.int32),
        pltpu.VMEM((_RING, _STEP, _D), jnp.float32),
        [pltpu.SemaphoreType.DMA] * _RING,
        [pltpu.SemaphoreType.DMA] * _RING,
    ],
    compiler_params=pltpu.CompilerParams(use_tc_tiling_on_sc=False),
)
def _emb_gather(idx_hbm, table_hbm, out_hbm, idx_v, bufs, gsems, wsems):
    wid = lax.axis_index("s") * _NC + lax.axis_index("c")
    base = wid * _PER_W
    pltpu.sync_copy(idx_hbm.at[pl.ds(base, _PER_W)], idx_v)

    def fire_gather(s, i):
        pltpu.async_copy(
            table_hbm.at[idx_v.at[pl.ds(s * _STEP, _STEP)]],
            bufs.at[i],
            gsems[i],
        )

    for i in range(_RING):
        fire_gather(i, i)

    def outer_body(k, carry):
        for i in range(_RING):
            s = k * _RING + i
            pltpu.make_async_copy(
                table_hbm.at[idx_v.at[pl.ds(s * _STEP, _STEP)]],
                bufs.at[i],
                gsems[i],
            ).wait()
            wcp = pltpu.async_copy(
                bufs.at[i],
                out_hbm.at[pl.ds(base + s * _STEP, _STEP)],
                wsems[i],
            )

            @pl.when(k < _NOUTER - 1)
            def _():
                wcp.wait()
                fire_gather(s + _RING, i)

        return carry

    lax.fori_loop(0, _NOUTER, outer_body, 0)

    for i in range(_RING):
        s = (_NOUTER - 1) * _RING + i
        pltpu.make_async_copy(
            bufs.at[i],
            out_hbm.at[pl.ds(base + s * _STEP, _STEP)],
            wsems[i],
        ).wait()


def kernel(word_ids, word_emb):
    idx = word_ids.reshape(_TOTAL)
    return _emb_gather(idx, word_emb).reshape(_B, _L, _D)
